# baseline (speedup 1.0000x reference)
import jax
import jax.numpy as jnp
from jax import lax
from jax.experimental import pallas as pl

_N = 5000
_B = 512
_NB = 10
_NT = _B * _NB
_TH = 0.7
_CW = 2560


def _overlap(px1, py1, px2, py2, pa, rows_ref, off, w):
    x1r = rows_ref[0:1, off:off + w]
    y1r = rows_ref[1:2, off:off + w]
    x2r = rows_ref[2:3, off:off + w]
    y2r = rows_ref[3:4, off:off + w]
    ar = rows_ref[5:6, off:off + w]
    ix1 = jnp.maximum(px1, x1r)
    iy1 = jnp.maximum(py1, y1r)
    ix2 = jnp.minimum(px2, x2r)
    iy2 = jnp.minimum(py2, y2r)
    inter = jnp.maximum(ix2 - ix1, 0.0) * jnp.maximum(iy2 - iy1, 0.0)
    union = pa + ar - inter
    return ((inter / union) > _TH).astype(jnp.float32)


def _nms_body(rows_ref, cols_ref, keep_ref):
    rid = lax.broadcasted_iota(jnp.int32, (_B, _B), 0)
    cid = lax.broadcasted_iota(jnp.int32, (_B, _B), 1)
    tri = (rid < cid).astype(jnp.float32)
    lcol = lax.broadcasted_iota(jnp.int32, (1, _CW), 1)

    def nms_pass(k_it):
        keep_ref[...] = jnp.ones((1, _NT), jnp.float32)
        acc = jnp.zeros((1, _B), jnp.float32)

        for i in range(_NB):
            base = i * _B
            c = cols_ref[i, :, :]
            px1 = c[:, 0:1]
            py1 = c[:, 1:2]
            px2 = c[:, 2:3]
            py2 = c[:, 3:4]
            pa = c[:, 5:6]

            rest = _NT - base
            widths = []
            while rest > 0:
                widths.append(min(_CW, rest))
                rest -= widths[-1]

            ov0 = _overlap(px1, py1, px2, py2, pa, rows_ref, base, widths[0])
            om = ov0[:, 0:_B] * tri
            pre = keep_ref[0:1, base:base + _B]

            def fix(kv, om=om, pre=pre):
                s = lax.dot_general(kv, om, (((1,), (0,)), ((), ())),
                                    preferred_element_type=jnp.float32)
                return pre * (s == 0.0).astype(jnp.float32)

            kv = lax.fori_loop(0, k_it, lambda _, kv: fix(kv), pre)
            kv_f = fix(kv)
            acc = acc + jnp.abs(kv_f - kv)
            keep_ref[0:1, base:base + _B] = kv_f

            off = base
            for ci, w in enumerate(widths):
                ov = ov0 if ci == 0 else _overlap(px1, py1, px2, py2, pa,
                                                  rows_ref, off, w)
                s_all = lax.dot_general(kv_f, ov, (((1,), (0,)), ((), ())),
                                        preferred_element_type=jnp.float32)
                sup = s_all > 0.0
                if ci == 0:
                    sup = sup & (lcol[:, 0:w] >= _B)
                keep_ref[0:1, off:off + w] = (
                    keep_ref[0:1, off:off + w] * (1.0 - sup.astype(jnp.float32)))
                off += w
        return acc

    def attempt_cond(carry):
        return carry[1]

    def attempt(carry):
        k_it, _ = carry
        acc = nms_pass(k_it)
        return (k_it * 2, jnp.any(acc > 0.0))

    lax.while_loop(attempt_cond, attempt, (jnp.int32(4), jnp.bool_(True)))


def _nms_keep(rows, cols):
    return pl.pallas_call(
        _nms_body,
        out_shape=jax.ShapeDtypeStruct((1, _NT), jnp.float32),
    )(rows, cols)


def kernel(boxes, scores):
    order = jnp.argsort(-scores)
    area = (boxes[:, 2] - boxes[:, 0]) * (boxes[:, 3] - boxes[:, 1])
    packed = jnp.concatenate(
        [boxes, scores[:, None], area[:, None], jnp.zeros((_N, 2), boxes.dtype)],
        axis=1)
    g = jnp.take(packed, order, axis=0)

    gp = jnp.pad(g, ((0, _NT - _N), (0, 0)))
    cols = gp.reshape(_NB, _B, 8)
    rows = gp.T

    keep = _nms_keep(rows, cols)
    out = g[:, 0:5] * keep[0, :_N, None]
    return out


# speedup vs baseline: 1.4605x; 1.4605x over previous
import jax
import jax.numpy as jnp
from jax import lax
from jax.experimental import pallas as pl

_N = 5000
_B = 512
_NB = 10
_NT = _B * _NB
_TH = 0.7
_CW = 2560


def _overlap(px1, py1, px2, py2, pa, rows_ref, off, w):
    x1r = rows_ref[0:1, off:off + w]
    y1r = rows_ref[1:2, off:off + w]
    x2r = rows_ref[2:3, off:off + w]
    y2r = rows_ref[3:4, off:off + w]
    ar = rows_ref[5:6, off:off + w]
    ix1 = jnp.maximum(px1, x1r)
    iy1 = jnp.maximum(py1, y1r)
    ix2 = jnp.minimum(px2, x2r)
    iy2 = jnp.minimum(py2, y2r)
    inter = jnp.maximum(ix2 - ix1, 0.0) * jnp.maximum(iy2 - iy1, 0.0)
    union = pa + ar - inter
    return ((inter / union) > _TH).astype(jnp.float32)


def _nms_body(rows_ref, cols_ref, keep_ref):
    rid = lax.broadcasted_iota(jnp.int32, (_B, _B), 0)
    cid = lax.broadcasted_iota(jnp.int32, (_B, _B), 1)
    tri = (rid < cid).astype(jnp.float32)
    lcol = lax.broadcasted_iota(jnp.int32, (1, _CW), 1)

    def nms_pass(exact):
        keep_ref[...] = jnp.ones((1, _NT), jnp.float32)
        acc = jnp.zeros((1, _B), jnp.float32)

        for i in range(_NB):
            base = i * _B
            c = cols_ref[i, :, :]
            px1 = c[:, 0:1]
            py1 = c[:, 1:2]
            px2 = c[:, 2:3]
            py2 = c[:, 3:4]
            pa = c[:, 5:6]

            rest = _NT - base
            widths = []
            while rest > 0:
                widths.append(min(_CW, rest))
                rest -= widths[-1]

            ov0 = _overlap(px1, py1, px2, py2, pa, rows_ref, base, widths[0])
            om = ov0[:, 0:_B] * tri
            pre = keep_ref[0:1, base:base + _B]

            def fix(kv, om=om, pre=pre):
                s = lax.dot_general(kv, om, (((1,), (0,)), ((), ())),
                                    preferred_element_type=jnp.float32)
                return pre * (s == 0.0).astype(jnp.float32)

            if exact:
                kv_a = fix(pre)
                kv = fix(kv_a)

                def fix_cond(carry):
                    return carry[1]

                def fix_body(carry, fix=fix):
                    nk = fix(carry[0])
                    return (nk, jnp.any(nk != carry[0]))

                kv_f, _ = lax.while_loop(fix_cond, fix_body,
                                         (kv, jnp.any(kv != kv_a)))
            else:
                kv = fix(fix(pre))
                kv_f = fix(kv)
                acc = acc + jnp.abs(kv_f - kv)
            keep_ref[0:1, base:base + _B] = kv_f

            off = base
            for ci, w in enumerate(widths):
                ov = ov0 if ci == 0 else _overlap(px1, py1, px2, py2, pa,
                                                  rows_ref, off, w)
                s_all = lax.dot_general(kv_f, ov, (((1,), (0,)), ((), ())),
                                        preferred_element_type=jnp.float32)
                sup = s_all > 0.0
                if ci == 0:
                    sup = sup & (lcol[:, 0:w] >= _B)
                keep_ref[0:1, off:off + w] = (
                    keep_ref[0:1, off:off + w] * (1.0 - sup.astype(jnp.float32)))
                off += w
        return acc

    acc = nms_pass(exact=False)

    @pl.when(jnp.any(acc > 0.0))
    def _redo():
        nms_pass(exact=True)


def _nms_keep(rows, cols):
    return pl.pallas_call(
        _nms_body,
        out_shape=jax.ShapeDtypeStruct((1, _NT), jnp.float32),
    )(rows, cols)


def kernel(boxes, scores):
    order = jnp.argsort(-scores)
    area = (boxes[:, 2] - boxes[:, 0]) * (boxes[:, 3] - boxes[:, 1])
    packed = jnp.concatenate(
        [boxes, scores[:, None], area[:, None], jnp.zeros((_N, 2), boxes.dtype)],
        axis=1)
    g = jnp.take(packed, order, axis=0)

    gp = jnp.pad(g, ((0, _NT - _N), (0, 0)))
    cols = gp.reshape(_NB, _B, 8)
    rows = gp.T

    keep = _nms_keep(rows, cols)
    out = g[:, 0:5] * keep[0, :_N, None]
    return out
